# baseline (speedup 1.0000x reference)
import jax
import jax.numpy as jnp
from jax.experimental import pallas as pl
from jax.experimental.pallas import tpu as pltpu

_NEG_POS_RATIO_MID = 3
_NEG_POS_RATIO_LOW = 3
_INT_MIN = -2147483648


def _lse_kernel(conf_ref, m0_ref, c1_ref):
    conf = conf_ref[0]
    confT = conf.T
    mx = jnp.max(confT, axis=0, keepdims=True)
    ex = jnp.exp(confT - mx)
    s = jnp.sum(ex, axis=0, keepdims=True)
    lse = jnp.log(s) + mx
    m0_ref[0] = lse - confT[0:1, :]
    c1_ref[0] = lse - confT[1:2, :]


def _select_kernel(m0_ref, c1_ref, lab_ref, mid_ref, low_ref, pred_ref,
                   gt_ref, lab4_ref, sl1_out_ref, cls_out_ref):
    m0 = m0_ref[...]
    c1 = c1_ref[...]
    lab = lab_ref[...]
    b, p = m0.shape

    pos = lab > 0
    ce = jnp.where(pos, c1, m0)
    mining = jnp.where(pos, -jnp.inf, m0)

    bits = jax.lax.bitcast_convert_type(mining, jnp.int32)
    key = jnp.where(bits >= 0, bits,
                    jnp.bitwise_xor(jnp.bitwise_not(bits), _INT_MIN))

    n_mid = jnp.sum((mid_ref[...] > 0).astype(jnp.int32), axis=1,
                    keepdims=True)
    n_low = jnp.sum((low_ref[...] > 0).astype(jnp.int32), axis=1,
                    keepdims=True)
    k = jnp.minimum(_NEG_POS_RATIO_MID * n_mid + _NEG_POS_RATIO_LOW * n_low,
                    p)

    def tbody(i, lo):
        cand = lo + jnp.left_shift(jnp.int32(1), 31 - i)
        cnt = jnp.sum((key >= cand).astype(jnp.int32), axis=1, keepdims=True)
        return jnp.where(cnt >= k, cand, lo)

    t = jax.lax.fori_loop(0, 32, tbody,
                          jnp.full((b, 1), _INT_MIN, jnp.int32))

    cnt_gt = jnp.sum((key > t).astype(jnp.int32), axis=1, keepdims=True)
    m = k - cnt_gt
    tie = key == t
    idx = jax.lax.broadcasted_iota(jnp.int32, (b, p), 1)

    def ibody(i, lo):
        cand = lo + jnp.left_shift(jnp.int32(1), 13 - i)
        f = jnp.sum((tie & (idx < cand)).astype(jnp.int32),
                    axis=1, keepdims=True)
        return jnp.where(f < m, cand, lo)

    loi = jax.lax.fori_loop(0, 14, ibody, jnp.zeros((b, 1), jnp.int32))
    istar = jnp.where(m > 0, loi + 1, 0)

    neg = (key > t) | (tie & (idx < istar))
    cls = jnp.sum(jnp.where(pos | neg, ce, 0.0))

    d = pred_ref[...] - gt_ref[...]
    ad = jnp.abs(d)
    sl1 = jnp.where(ad < 1.0, 0.5 * d * d, ad - 0.5)
    sl1_sum = jnp.sum(jnp.where(lab4_ref[...] > 0, sl1, 0.0))

    npos = jnp.sum(pos.astype(jnp.float32)) + 1e-06
    sl1_out_ref[...] = (sl1_sum / npos).reshape(1, 1)
    cls_out_ref[...] = (cls / npos).reshape(1, 1)


@jax.jit
def kernel(confidence, predicted_locations, labels, labels_mid, labels_low,
           gt_locations):
    bsz, p, c = confidence.shape

    n_sub = 4 if p % 4 == 0 else 1
    pb = p // n_sub
    g = bsz * n_sub
    confr = confidence.reshape(g, pb, c)

    m0, c1 = pl.pallas_call(
        _lse_kernel,
        grid=(g,),
        in_specs=[
            pl.BlockSpec((1, pb, c), lambda i: (i, 0, 0)),
        ],
        out_specs=[
            pl.BlockSpec((1, 1, pb), lambda i: (i, 0, 0)),
            pl.BlockSpec((1, 1, pb), lambda i: (i, 0, 0)),
        ],
        out_shape=[
            jax.ShapeDtypeStruct((g, 1, pb), jnp.float32),
            jax.ShapeDtypeStruct((g, 1, pb), jnp.float32),
        ],
        compiler_params=pltpu.CompilerParams(
            dimension_semantics=("parallel",)),
    )(confr)

    lab = labels.astype(jnp.int32)
    lab4 = jnp.repeat(lab, 4, axis=1)

    sl1_out, cls_out = pl.pallas_call(
        _select_kernel,
        out_shape=[
            jax.ShapeDtypeStruct((1, 1), jnp.float32),
            jax.ShapeDtypeStruct((1, 1), jnp.float32),
        ],
    )(m0.reshape(bsz, p), c1.reshape(bsz, p), lab,
      labels_mid.astype(jnp.int32), labels_low.astype(jnp.int32),
      predicted_locations.reshape(bsz, 4 * p),
      gt_locations.reshape(bsz, 4 * p), lab4)

    return (sl1_out[0, 0], cls_out[0, 0])


# speedup vs baseline: 1.6966x; 1.6966x over previous
import jax
import jax.numpy as jnp
from jax.experimental import pallas as pl
from jax.experimental.pallas import tpu as pltpu

_NEG_POS_RATIO_MID = 3
_NEG_POS_RATIO_LOW = 3
_INT_MIN = -2147483648


def _lse_kernel(conf_ref, m0_ref, c1_ref):
    conf = conf_ref[0]
    confT = conf.T
    mx = jnp.max(confT, axis=0, keepdims=True)
    ex = jnp.exp(confT - mx)
    s = jnp.sum(ex, axis=0, keepdims=True)
    lse = jnp.log(s) + mx
    m0_ref[0] = lse - confT[0:1, :]
    c1_ref[0] = lse - confT[1:2, :]


def _select_kernel(m0_ref, c1_ref, lab_ref, mid_ref, low_ref, pred_ref,
                   gt_ref, lab4_ref, sl1_out_ref, cls_out_ref):
    m0 = m0_ref[...]
    c1 = c1_ref[...]
    lab = lab_ref[...]
    b, p = m0.shape

    pos = lab > 0
    ce = jnp.where(pos, c1, m0)
    mining = jnp.where(pos, -jnp.inf, m0)

    bits = jax.lax.bitcast_convert_type(mining, jnp.int32)
    key = jnp.where(bits >= 0, bits,
                    jnp.bitwise_xor(jnp.bitwise_not(bits), _INT_MIN))

    n_mid = jnp.sum((mid_ref[...] > 0).astype(jnp.int32), axis=1,
                    keepdims=True)
    n_low = jnp.sum((low_ref[...] > 0).astype(jnp.int32), axis=1,
                    keepdims=True)
    k = jnp.minimum(_NEG_POS_RATIO_MID * n_mid + _NEG_POS_RATIO_LOW * n_low,
                    p)

    def tbody(i, lo):
        cand = lo + jnp.left_shift(jnp.int32(1), 31 - i)
        cnt = jnp.sum((key >= cand).astype(jnp.int32), axis=1, keepdims=True)
        return jnp.where(cnt >= k, cand, lo)

    t = jax.lax.fori_loop(0, 32, tbody,
                          jnp.full((b, 1), _INT_MIN, jnp.int32))

    cnt_gt = jnp.sum((key > t).astype(jnp.int32), axis=1, keepdims=True)
    m = k - cnt_gt
    tie = key == t
    idx = jax.lax.broadcasted_iota(jnp.int32, (b, p), 1)

    def ibody(i, lo):
        cand = lo + jnp.left_shift(jnp.int32(1), 13 - i)
        f = jnp.sum((tie & (idx < cand)).astype(jnp.int32),
                    axis=1, keepdims=True)
        return jnp.where(f < m, cand, lo)

    loi = jax.lax.fori_loop(0, 14, ibody, jnp.zeros((b, 1), jnp.int32))
    istar = jnp.where(m > 0, loi + 1, 0)

    neg = (key > t) | (tie & (idx < istar))
    cls = jnp.sum(jnp.where(pos | neg, ce, 0.0))

    d = pred_ref[...] - gt_ref[...]
    ad = jnp.abs(d)
    sl1 = jnp.where(ad < 1.0, 0.5 * d * d, ad - 0.5)
    sl1_sum = jnp.sum(jnp.where(lab4_ref[...] > 0, sl1, 0.0))

    npos = jnp.sum(pos.astype(jnp.float32)) + 1e-06
    sl1_out_ref[...] = (sl1_sum / npos).reshape(1, 1)
    cls_out_ref[...] = (cls / npos).reshape(1, 1)


@jax.jit
def kernel(confidence, predicted_locations, labels, labels_mid, labels_low,
           gt_locations):
    bsz, p, c = confidence.shape

    n_sub = 1
    pb = p // n_sub
    g = bsz * n_sub
    confr = confidence.reshape(g, pb, c)

    m0, c1 = pl.pallas_call(
        _lse_kernel,
        grid=(g,),
        in_specs=[
            pl.BlockSpec((1, pb, c), lambda i: (i, 0, 0)),
        ],
        out_specs=[
            pl.BlockSpec((1, 1, pb), lambda i: (i, 0, 0)),
            pl.BlockSpec((1, 1, pb), lambda i: (i, 0, 0)),
        ],
        out_shape=[
            jax.ShapeDtypeStruct((g, 1, pb), jnp.float32),
            jax.ShapeDtypeStruct((g, 1, pb), jnp.float32),
        ],
        compiler_params=pltpu.CompilerParams(
            dimension_semantics=("parallel",)),
    )(confr)

    lab = labels.astype(jnp.int32)
    lab4 = jnp.repeat(lab, 4, axis=1)

    sl1_out, cls_out = pl.pallas_call(
        _select_kernel,
        out_shape=[
            jax.ShapeDtypeStruct((1, 1), jnp.float32),
            jax.ShapeDtypeStruct((1, 1), jnp.float32),
        ],
    )(m0.reshape(bsz, p), c1.reshape(bsz, p), lab,
      labels_mid.astype(jnp.int32), labels_low.astype(jnp.int32),
      predicted_locations.reshape(bsz, 4 * p),
      gt_locations.reshape(bsz, 4 * p), lab4)

    return (sl1_out[0, 0], cls_out[0, 0])
